# Initial kernel scaffold; baseline (speedup 1.0000x reference)
import jax
import jax.numpy as jnp
from jax.experimental import pallas as pl


def kernel(X0, X1_idx, X2_idx, L0_rows, L0_cols, L0_vals, L1_rows, L1_cols, L1_vals, L1u_rows, L1u_cols, L1u_vals, L1d_rows, L1d_cols, L1d_vals, L2_rows, L2_cols, L2_vals, B1_rows, B1_cols, B1_vals, B2_rows, B2_cols, B2_vals, Wn_u, bn_u, Wn_d, bn_d, Wn_p, bn_p, We_u, be_u, We_d, be_d, We_p, be_p, Wt_u, bt_u, Wt_d, bt_d, Wt_p, bt_p, W_tri, b_tri, prelu_w):
    raise NotImplementedError("write your pallas kernel here")


# speedup vs baseline: 1.1771x; 1.1771x over previous
import functools

import jax
import jax.numpy as jnp
from jax.experimental import pallas as pl
from jax.experimental.pallas import tpu as pltpu


def _mm_body(x_ref, w_ref, b_ref, o_ref):
    o_ref[...] = (
        jnp.dot(x_ref[...], w_ref[...], preferred_element_type=jnp.float32)
        + b_ref[...]
    )


def _mm(X, W, b, block=1024):
    N, K = X.shape
    F = W.shape[1]
    grid = (pl.cdiv(N, block),)
    return pl.pallas_call(
        _mm_body,
        grid=grid,
        in_specs=[
            pl.BlockSpec((block, K), lambda i: (i, 0)),
            pl.BlockSpec((K, F), lambda i: (0, 0)),
            pl.BlockSpec((1, F), lambda i: (0, 0)),
        ],
        out_specs=pl.BlockSpec((block, F), lambda i: (i, 0)),
        out_shape=jax.ShapeDtypeStruct((N, F), jnp.float32),
    )(X, W, b.reshape(1, F))


def _spmm(rows, cols, vals, n_rows, X):
    return jnp.zeros((n_rows, X.shape[1]), X.dtype).at[rows].add(
        vals[:, None] * X[cols]
    )


def kernel(X0, X1_idx, X2_idx, L0_rows, L0_cols, L0_vals, L1_rows, L1_cols, L1_vals, L1u_rows, L1u_cols, L1u_vals, L1d_rows, L1d_cols, L1d_vals, L2_rows, L2_cols, L2_vals, B1_rows, B1_cols, B1_vals, B2_rows, B2_cols, B2_vals, Wn_u, bn_u, Wn_d, bn_d, Wn_p, bn_p, We_u, be_u, We_d, be_d, We_p, be_p, Wt_u, bt_u, Wt_d, bt_d, Wt_p, bt_p, W_tri, b_tri, prelu_w):
    N0, D = X0.shape
    N1 = X1_idx.shape[0]
    N2 = X2_idx.shape[0]

    def prelu(x):
        return jnp.where(x >= 0, x, prelu_w * x)

    X0b = jnp.where(X0 != 0, 1.0, 0.0).astype(jnp.float32)

    Y0 = _mm(X0b, Wn_p + Wn_d, bn_p + bn_d)
    X0h = prelu(_spmm(L0_rows, L0_cols, L0_vals, N0, Y0))

    X1f = X0b[X1_idx[:, 0]] * X0b[X1_idx[:, 1]]
    We = jnp.concatenate([We_p, We_u, We_d], axis=1)
    be = jnp.concatenate([be_p, be_u, be_d], axis=0)
    Y1 = _mm(X1f, We, be)
    H1 = (
        _spmm(L1_rows, L1_cols, L1_vals, N1, Y1[:, :D])
        + _spmm(L1u_rows, L1u_cols, L1u_vals, N1, Y1[:, D:2 * D])
        + _spmm(L1d_rows, L1d_cols, L1d_vals, N1, Y1[:, 2 * D:])
    )
    X1h = prelu(H1)

    X2f = X0b[X2_idx[:, 0]] * X0b[X2_idx[:, 1]] * X0b[X2_idx[:, 2]]
    Y2 = _mm(X2f, Wt_p + Wt_u, bt_p + bt_u)
    X2h = prelu(_spmm(L2_rows, L2_cols, L2_vals, N2, Y2))

    T = _spmm(B2_rows, B2_cols, B2_vals, N1, X2h)
    tri = _mm(T, W_tri, b_tri)
    S = _spmm(B1_rows, B1_cols, B1_vals, N0, X1h + tri)
    return (X0h + S) / 3.0
